# baseline (speedup 1.0000x reference)
import functools

import jax
import jax.numpy as jnp
from jax import lax
from jax.experimental import pallas as pl
from jax.experimental.pallas import tpu as pltpu
from jax.experimental.pallas import tpu_sc as plsc

NC = 2
NS = 16
W = NC * NS
L = 16

T = 256
T8 = T + 8
IDB = T + 48
NBUF = 3
K = 64
TS = 2000

NEG_INF = float("-inf")


def _make(N, D, S):
    assert N % W == 0
    C = N // W
    assert C % TS == 0 and TS % L == 0 and C % 8 == 0
    KD = D // L

    mesh = plsc.VectorSubcoreMesh(
        core_axis_name="c", subcore_axis_name="s", num_cores=NC, num_subcores=NS
    )

    def al8(x):
        return pl.multiple_of(x, 8)

    def ds16(off):
        return pl.ds(pl.multiple_of(off, L), L)


    @functools.partial(
        pl.kernel,
        out_type=jax.ShapeDtypeStruct((S * D,), jnp.float32),
        mesh=mesh,
        scratch_types=[
            pltpu.VMEM((NBUF * T8 * D,), jnp.float32),
            pltpu.VMEM((NBUF * IDB,), jnp.int32),
            pltpu.VMEM((TS + 24,), jnp.int32),
            pltpu.VMEM((K * D,), jnp.float32),
            pltpu.VMEM((D,), jnp.float32),
            pltpu.SemaphoreType.DMA,
            pltpu.SemaphoreType.DMA,
        ],
    )
    def seg_max(feat, ids, out, vbuf, idbuf, sbuf, stag, aref, semf, semi):
        wid = lax.axis_index("s") * NC + lax.axis_index("c")

        def lane0(ref, idx):
            return ref[pl.ds(idx, L)][0]

        def minf():
            return jnp.full((L,), NEG_INF, jnp.float32)

        def clear_stag():
            def cbody(q, _):
                for k in range(KD):
                    stag[ds16(q * D + L * k)] = minf()
                return 0

            lax.fori_loop(0, K, cbody, 0)

        def clear_aref():
            for k in range(KD):
                aref[pl.ds(L * k, L)] = minf()

        def find_start(b):
            nw = (N - b) // TS

            def wbody(jw, carry):
                found, prev = carry

                def do():
                    base = b + jw * TS
                    pltpu.sync_copy(
                        ids.at[pl.ds(al8(base - 8), TS + 8)],
                        sbuf.at[pl.ds(0, TS + 8)],
                    )

                    def gbody(g, c2):
                        found2, prev2 = c2

                        def gdo():
                            e0 = lane0(sbuf, 7 + L * g)
                            e1 = lane0(sbuf, 7 + L * g + L)

                            def scan16():
                                def rbody(i, c3):
                                    f3, p3 = c3

                                    def rdo():
                                        a = lane0(sbuf, 7 + L * g + i)
                                        c = lane0(sbuf, 8 + L * g + i)
                                        hit = a != c
                                        return (
                                            jnp.where(hit, base + L * g + i, f3),
                                            jnp.where(hit, a, p3),
                                        )

                                    return lax.cond(f3 < 0, rdo, lambda: c3)

                                return lax.fori_loop(0, L, rbody, (found2, prev2))

                            return lax.cond(e0 != e1, scan16, lambda: (found2, prev2))

                        return lax.cond(found2 < 0, gdo, lambda: (found2, prev2))

                    found2, prev2 = lax.fori_loop(0, TS // L, gbody, (found, prev))
                    last = lane0(sbuf, TS + 7)
                    prev3 = jnp.where(
                        jnp.logical_and(found2 < 0, jw == nw - 1), last, prev2
                    )
                    return found2, prev3

                return lax.cond(found < 0, do, lambda: (found, prev))

            found, prev = lax.fori_loop(
                0, nw, wbody, (jnp.int32(-1), jnp.int32(-1))
            )
            return jnp.where(found < 0, jnp.int32(N), found), prev

        start, prevlo = lax.cond(
            wid == 0,
            lambda: (jnp.int32(0), jnp.int32(-1)),
            lambda: find_start(wid * C),
        )
        start_next, prevhi = lax.cond(
            wid == W - 1,
            lambda: (jnp.int32(N), jnp.int32(S - 1)),
            lambda: find_start((wid + 1) * C),
        )
        lo = prevlo + 1
        lo_next = prevhi + 1
        nrows = start_next - start

        clear_stag()
        clear_aref()
        ntiles = (nrows + T - 1) // T

        def emit(cur, fb):
            nf = (cur - fb) // K

            def flush(i, fb2):
                pltpu.sync_copy(stag, out.at[pl.ds(al8(fb2 * D), K * D)])
                clear_stag()
                return fb2 + K

            fb = lax.fori_loop(0, nf, flush, fb)
            q = cur - fb
            for k in range(KD):
                stag[ds16(q * D + L * k)] = aref[pl.ds(L * k, L)]
                aref[pl.ds(L * k, L)] = minf()
            return fb

        def tile_src(g):
            row0 = start + g * T
            row0a = al8(jnp.minimum(8 * (row0 // 8), N - T8))
            al = al8(jnp.minimum(8 * (row0 // 8), N - (T + 16)))
            return row0, row0a, al

        def issue(g):
            row0, row0a, al = tile_src(g)
            p = g % NBUF
            pltpu.async_copy(
                feat.at[pl.ds(al8(row0a * D), T8 * D)],
                vbuf.at[pl.ds(al8(p * (T8 * D)), T8 * D)],
                semf,
            )
            pltpu.async_copy(
                ids.at[pl.ds(al, T + 16)],
                idbuf.at[pl.ds(al8(p * IDB), T + 16)],
                semi,
            )
            return jnp.int32(0)

        def tile_body(g, carry):
            cur_c, fb_c = carry
            row0, row0a, al = tile_src(g)
            trows = jnp.minimum(T, start_next - row0)
            doff = row0 - row0a
            joff = row0 - al
            p = g % NBUF
            vb0 = p * (T8 * D)
            ib0 = p * IDB
            pltpu.make_async_copy(
                feat.at[pl.ds(0, T8 * D)],
                vbuf.at[pl.ds(al8(vb0), T8 * D)],
                semf,
            ).wait()
            pltpu.make_async_copy(
                ids.at[pl.ds(0, T + 16)],
                idbuf.at[pl.ds(al8(ib0), T + 16)],
                semi,
            ).wait()
            nxt = g + NBUF - 1
            lax.cond(nxt < ntiles, lambda: issue(nxt), lambda: jnp.int32(0))

            def acc_row(row):
                for k in range(KD):
                    aref[pl.ds(L * k, L)] = jnp.maximum(
                        aref[pl.ds(L * k, L)],
                        vbuf[ds16(vb0 + row * D + L * k)],
                    )

            def row_step(r, cur, fb):
                seg = lane0(idbuf, ib0 + joff + r)

                def boundary():
                    fb2 = emit(cur, fb)
                    return seg, fb2

                cur, fb = lax.cond(seg != cur, boundary, lambda: (cur, fb))
                acc_row(doff + r)
                return cur, fb

            def group_body(gg, c2):
                cur, fb = c2
                p0 = gg * L
                vids = idbuf[pl.ds(ib0 + joff + p0, L)]
                glast = vids[L - 1]

                def slow():
                    def rbody(r, c3):
                        return row_step(p0 + r, *c3)

                    return lax.fori_loop(0, L, rbody, (cur, fb))

                def fast():
                    acc = [aref[pl.ds(L * k, L)] for k in range(KD)]
                    for j in range(L):
                        rowb = doff + p0 + j
                        for k in range(KD):
                            acc[k] = jnp.maximum(
                                acc[k], vbuf[ds16(vb0 + rowb * D + L * k)]
                            )
                    q = cur - fb
                    for k in range(KD):
                        aref[pl.ds(L * k, L)] = acc[k]
                        stag[ds16(q * D + L * k)] = acc[k]
                    return cur, fb

                def mixed():
                    segs = [vids[j] for j in range(L - 1)] + [glast]
                    acc = [aref[pl.ds(L * k, L)] for k in range(KD)]
                    qp = cur - fb
                    sp = cur
                    for k in range(KD):
                        stag[ds16(qp * D + L * k)] = acc[k]
                    for r in range(L):
                        qr = segs[r] - fb
                        pf = jnp.where(
                            segs[r] == sp, jnp.float32(0.0), jnp.float32(jnp.inf)
                        )
                        pv = jnp.broadcast_to(pf, (L,))
                        rowb = doff + p0 + r
                        for k in range(KD):
                            acc[k] = jnp.maximum(
                                acc[k] - pv,
                                vbuf[ds16(vb0 + rowb * D + L * k)],
                            )
                        for k in range(KD):
                            stag[ds16(qr * D + L * k)] = acc[k]
                        qp = qr
                        sp = segs[r]
                    for k in range(KD):
                        aref[pl.ds(L * k, L)] = acc[k]
                    return glast, fb

                def no_flush():
                    return lax.cond(glast == cur, fast, mixed)

                return lax.cond(glast - fb >= K, slow, no_flush)

            ngroups = trows // L
            cur_c, fb_c = lax.fori_loop(0, ngroups, group_body, (cur_c, fb_c))

            def rem_body(r, c2):
                return row_step(r, *c2)

            cur_c, fb_c = lax.fori_loop(ngroups * L, trows, rem_body, (cur_c, fb_c))
            return cur_c, fb_c

        def seed():
            al0 = al8(jnp.minimum(8 * (start // 8), N - (T + 16)))
            pltpu.sync_copy(ids.at[pl.ds(al0, T + 16)], sbuf.at[pl.ds(0, T + 16)])
            return lane0(sbuf, start - al0)

        cur0 = lax.cond(nrows > 0, seed, lambda: prevlo + 1)
        for _pr in range(NBUF - 1):
            lax.cond(_pr < ntiles, lambda g=_pr: issue(g), lambda: jnp.int32(0))
        cur, fb = lax.fori_loop(0, ntiles, tile_body, (cur0, lo))
        fb = lax.cond(nrows > 0, lambda: emit(cur, fb), lambda: fb)

        nf = (lo_next - fb) // K

        def tflush(i, fb2):
            pltpu.sync_copy(stag, out.at[pl.ds(al8(fb2 * D), K * D)])
            clear_stag()
            return fb2 + K

        fb = lax.fori_loop(0, nf, tflush, fb)

        def trow(i, _):
            pltpu.sync_copy(
                stag.at[pl.ds(al8(i * D), D)],
                out.at[pl.ds(al8((fb + i) * D), D)],
            )
            return 0

        lax.fori_loop(0, lo_next - fb, trow, 0)

    return seg_max


def kernel(features, segment_ids):
    N, D = features.shape
    S = 10000
    flat = _make(N, D, S)(features.reshape(N * D), segment_ids)
    return flat.reshape(S, D)


# speedup vs baseline: 1.0021x; 1.0021x over previous
import functools

import jax
import jax.numpy as jnp
from jax import lax
from jax.experimental import pallas as pl
from jax.experimental.pallas import tpu as pltpu
from jax.experimental.pallas import tpu_sc as plsc

NC = 2
NS = 16
W = NC * NS
L = 16

T = 384
T8 = T + 8
IDB = T + 48
K = 64
TS = 2000

NEG_INF = float("-inf")


def _make(N, D, S):
    assert N % W == 0
    C = N // W
    assert C % TS == 0 and TS % L == 0 and C % 8 == 0
    KD = D // L

    mesh = plsc.VectorSubcoreMesh(
        core_axis_name="c", subcore_axis_name="s", num_cores=NC, num_subcores=NS
    )

    def al8(x):
        return pl.multiple_of(x, 8)

    def ds16(off):
        return pl.ds(pl.multiple_of(off, L), L)


    @functools.partial(
        pl.kernel,
        out_type=jax.ShapeDtypeStruct((S * D,), jnp.float32),
        mesh=mesh,
        scratch_types=[
            pltpu.VMEM((2 * T8 * D,), jnp.float32),
            pltpu.VMEM((2 * IDB,), jnp.int32),
            pltpu.VMEM((TS + 24,), jnp.int32),
            pltpu.VMEM((K * D,), jnp.float32),
            pltpu.VMEM((D,), jnp.float32),
            pltpu.SemaphoreType.DMA,
            pltpu.SemaphoreType.DMA,
        ],
    )
    def seg_max(feat, ids, out, vbuf, idbuf, sbuf, stag, aref, semf, semi):
        wid = lax.axis_index("s") * NC + lax.axis_index("c")

        def lane0(ref, idx):
            return ref[pl.ds(idx, L)][0]

        def minf():
            return jnp.full((L,), NEG_INF, jnp.float32)

        def clear_stag():
            def cbody(q, _):
                for k in range(KD):
                    stag[ds16(q * D + L * k)] = minf()
                return 0

            lax.fori_loop(0, K, cbody, 0)

        def clear_aref():
            for k in range(KD):
                aref[pl.ds(L * k, L)] = minf()

        def find_start(b):
            nw = (N - b) // TS

            def wbody(jw, carry):
                found, prev = carry

                def do():
                    base = b + jw * TS
                    pltpu.sync_copy(
                        ids.at[pl.ds(al8(base - 8), TS + 8)],
                        sbuf.at[pl.ds(0, TS + 8)],
                    )

                    def gbody(g, c2):
                        found2, prev2 = c2

                        def gdo():
                            e0 = lane0(sbuf, 7 + L * g)
                            e1 = lane0(sbuf, 7 + L * g + L)

                            def scan16():
                                def rbody(i, c3):
                                    f3, p3 = c3

                                    def rdo():
                                        a = lane0(sbuf, 7 + L * g + i)
                                        c = lane0(sbuf, 8 + L * g + i)
                                        hit = a != c
                                        return (
                                            jnp.where(hit, base + L * g + i, f3),
                                            jnp.where(hit, a, p3),
                                        )

                                    return lax.cond(f3 < 0, rdo, lambda: c3)

                                return lax.fori_loop(0, L, rbody, (found2, prev2))

                            return lax.cond(e0 != e1, scan16, lambda: (found2, prev2))

                        return lax.cond(found2 < 0, gdo, lambda: (found2, prev2))

                    found2, prev2 = lax.fori_loop(0, TS // L, gbody, (found, prev))
                    last = lane0(sbuf, TS + 7)
                    prev3 = jnp.where(
                        jnp.logical_and(found2 < 0, jw == nw - 1), last, prev2
                    )
                    return found2, prev3

                return lax.cond(found < 0, do, lambda: (found, prev))

            found, prev = lax.fori_loop(
                0, nw, wbody, (jnp.int32(-1), jnp.int32(-1))
            )
            return jnp.where(found < 0, jnp.int32(N), found), prev

        start, prevlo = lax.cond(
            wid == 0,
            lambda: (jnp.int32(0), jnp.int32(-1)),
            lambda: find_start(wid * C),
        )
        start_next, prevhi = lax.cond(
            wid == W - 1,
            lambda: (jnp.int32(N), jnp.int32(S - 1)),
            lambda: find_start((wid + 1) * C),
        )
        lo = prevlo + 1
        lo_next = prevhi + 1
        nrows = start_next - start

        clear_stag()
        clear_aref()
        ntiles = (nrows + T - 1) // T

        def emit(cur, fb):
            nf = (cur - fb) // K

            def flush(i, fb2):
                pltpu.sync_copy(stag, out.at[pl.ds(al8(fb2 * D), K * D)])
                clear_stag()
                return fb2 + K

            fb = lax.fori_loop(0, nf, flush, fb)
            q = cur - fb
            for k in range(KD):
                stag[ds16(q * D + L * k)] = aref[pl.ds(L * k, L)]
                aref[pl.ds(L * k, L)] = minf()
            return fb

        def tile_src(g):
            row0 = start + g * T
            row0a = al8(jnp.minimum(8 * (row0 // 8), N - T8))
            al = al8(jnp.minimum(8 * (row0 // 8), N - (T + 16)))
            return row0, row0a, al

        def issue(g):
            row0, row0a, al = tile_src(g)
            p = g % 2
            pltpu.async_copy(
                feat.at[pl.ds(al8(row0a * D), T8 * D)],
                vbuf.at[pl.ds(al8(p * (T8 * D)), T8 * D)],
                semf,
            )
            pltpu.async_copy(
                ids.at[pl.ds(al, T + 16)],
                idbuf.at[pl.ds(al8(p * IDB), T + 16)],
                semi,
            )
            return jnp.int32(0)

        def tile_body(g, carry):
            cur_c, fb_c = carry
            row0, row0a, al = tile_src(g)
            trows = jnp.minimum(T, start_next - row0)
            doff = row0 - row0a
            joff = row0 - al
            p = g % 2
            vb0 = p * (T8 * D)
            ib0 = p * IDB
            pltpu.make_async_copy(
                feat.at[pl.ds(0, T8 * D)],
                vbuf.at[pl.ds(al8(vb0), T8 * D)],
                semf,
            ).wait()
            pltpu.make_async_copy(
                ids.at[pl.ds(0, T + 16)],
                idbuf.at[pl.ds(al8(ib0), T + 16)],
                semi,
            ).wait()
            lax.cond(g + 1 < ntiles, lambda: issue(g + 1), lambda: jnp.int32(0))

            def acc_row(row):
                for k in range(KD):
                    aref[pl.ds(L * k, L)] = jnp.maximum(
                        aref[pl.ds(L * k, L)],
                        vbuf[ds16(vb0 + row * D + L * k)],
                    )

            def row_step(r, cur, fb):
                seg = lane0(idbuf, ib0 + joff + r)

                def boundary():
                    fb2 = emit(cur, fb)
                    return seg, fb2

                cur, fb = lax.cond(seg != cur, boundary, lambda: (cur, fb))
                acc_row(doff + r)
                return cur, fb

            def group_body(gg, c2):
                cur, fb = c2
                p0 = gg * L
                vids = idbuf[pl.ds(ib0 + joff + p0, L)]
                glast = vids[L - 1]

                def slow():
                    def rbody(r, c3):
                        return row_step(p0 + r, *c3)

                    return lax.fori_loop(0, L, rbody, (cur, fb))

                def fast():
                    acc = [aref[pl.ds(L * k, L)] for k in range(KD)]
                    for j in range(L):
                        rowb = doff + p0 + j
                        for k in range(KD):
                            acc[k] = jnp.maximum(
                                acc[k], vbuf[ds16(vb0 + rowb * D + L * k)]
                            )
                    q = cur - fb
                    for k in range(KD):
                        aref[pl.ds(L * k, L)] = acc[k]
                        stag[ds16(q * D + L * k)] = acc[k]
                    return cur, fb

                def mixed():
                    segs = [vids[j] for j in range(L - 1)] + [glast]
                    acc = [aref[pl.ds(L * k, L)] for k in range(KD)]
                    qp = cur - fb
                    sp = cur
                    for k in range(KD):
                        stag[ds16(qp * D + L * k)] = acc[k]
                    for r in range(L):
                        qr = segs[r] - fb
                        pf = jnp.where(
                            segs[r] == sp, jnp.float32(0.0), jnp.float32(jnp.inf)
                        )
                        pv = jnp.broadcast_to(pf, (L,))
                        rowb = doff + p0 + r
                        for k in range(KD):
                            acc[k] = jnp.maximum(
                                acc[k] - pv,
                                vbuf[ds16(vb0 + rowb * D + L * k)],
                            )
                        for k in range(KD):
                            stag[ds16(qr * D + L * k)] = acc[k]
                        qp = qr
                        sp = segs[r]
                    for k in range(KD):
                        aref[pl.ds(L * k, L)] = acc[k]
                    return glast, fb

                def no_flush():
                    return lax.cond(glast == cur, fast, mixed)

                return lax.cond(glast - fb >= K, slow, no_flush)

            ngroups = trows // L
            cur_c, fb_c = lax.fori_loop(0, ngroups, group_body, (cur_c, fb_c))

            def rem_body(r, c2):
                return row_step(r, *c2)

            cur_c, fb_c = lax.fori_loop(ngroups * L, trows, rem_body, (cur_c, fb_c))
            return cur_c, fb_c

        def seed():
            al0 = al8(jnp.minimum(8 * (start // 8), N - (T + 16)))
            pltpu.sync_copy(ids.at[pl.ds(al0, T + 16)], sbuf.at[pl.ds(0, T + 16)])
            return lane0(sbuf, start - al0)

        cur0 = lax.cond(nrows > 0, seed, lambda: prevlo + 1)
        lax.cond(ntiles > 0, lambda: issue(0), lambda: jnp.int32(0))
        cur, fb = lax.fori_loop(0, ntiles, tile_body, (cur0, lo))
        fb = lax.cond(nrows > 0, lambda: emit(cur, fb), lambda: fb)

        nf = (lo_next - fb) // K

        def tflush(i, fb2):
            pltpu.sync_copy(stag, out.at[pl.ds(al8(fb2 * D), K * D)])
            clear_stag()
            return fb2 + K

        fb = lax.fori_loop(0, nf, tflush, fb)

        def trow(i, _):
            pltpu.sync_copy(
                stag.at[pl.ds(al8(i * D), D)],
                out.at[pl.ds(al8((fb + i) * D), D)],
            )
            return 0

        lax.fori_loop(0, lo_next - fb, trow, 0)

    return seg_max


def kernel(features, segment_ids):
    N, D = features.shape
    S = 10000
    flat = _make(N, D, S)(features.reshape(N * D), segment_ids)
    return flat.reshape(S, D)
